# Initial kernel scaffold; baseline (speedup 1.0000x reference)
import jax
import jax.numpy as jnp
from jax.experimental import pallas as pl


def kernel(x, table, W, b):
    raise NotImplementedError("write your pallas kernel here")


# speedup vs baseline: 66.7030x; 66.7030x over previous
import functools

import jax
import jax.numpy as jnp
from jax import lax
from jax.experimental import pallas as pl
from jax.experimental.pallas import tpu as pltpu
from jax.experimental.pallas import tpu_sc as plsc

B = 4096
L = 200
V = 128
D = 32
NC = 2
NS = 16
LANES = 16
NW = NC * NS
BPW = B // NW
G = BPW // LANES


def _sc_body(x_hbm, tab_hbm, w_hbm, b_hbm, out_hbm,
             x_vm, tab_vm, w_vm, b_vm, v_vm, out_vm):
    cid = lax.axis_index("c")
    sid = lax.axis_index("s")
    wid = sid * NC + cid

    pltpu.sync_copy(x_hbm.at[pl.ds(wid * (BPW * L), BPW * L)], x_vm)
    pltpu.sync_copy(tab_hbm, tab_vm)
    pltpu.sync_copy(w_hbm, w_vm)
    pltpu.sync_copy(b_hbm, b_vm)

    lanes = lax.iota(jnp.int32, LANES)

    for g in range(V // LANES):
        rowbase = (g * LANES + lanes) * D
        acc = jnp.zeros((LANES,), jnp.float32)
        for dd in range(D):
            col = plsc.load_gather(tab_vm, [rowbase + dd])
            wsplat = w_vm[pl.ds(dd * LANES, LANES)]
            acc = acc + col * wsplat
        v_vm[pl.ds(g * LANES, LANES)] = acc

    rowoff = lanes * L

    def body(j, accs):
        nxt = []
        for g in range(G):
            idx = rowoff + (g * (LANES * L) + j)
            xv = plsc.load_gather(x_vm, [idx])
            nxt.append(accs[g] + plsc.load_gather(v_vm, [xv]))
        return tuple(nxt)

    zero = jnp.zeros((LANES,), jnp.float32)
    accs = lax.fori_loop(0, L, body, tuple(zero for _ in range(G)))

    bvec = b_vm[...]
    for g in range(G):
        z = accs[g] * (1.0 / L) + bvec
        out_vm[pl.ds(g * LANES, LANES)] = 1.0 / (1.0 + jnp.exp(-z))

    pltpu.sync_copy(out_vm, out_hbm.at[pl.ds(wid * BPW, BPW)])


_tt_call = functools.partial(
    pl.kernel,
    out_type=jax.ShapeDtypeStruct((B,), jnp.float32),
    mesh=plsc.VectorSubcoreMesh(core_axis_name="c", subcore_axis_name="s"),
    compiler_params=pltpu.CompilerParams(needs_layout_passes=False),
    scratch_types=[
        pltpu.VMEM((BPW * L,), jnp.int32),
        pltpu.VMEM((V * D,), jnp.float32),
        pltpu.VMEM((D * LANES,), jnp.float32),
        pltpu.VMEM((LANES,), jnp.float32),
        pltpu.VMEM((V,), jnp.float32),
        pltpu.VMEM((BPW,), jnp.float32),
    ],
)(_sc_body)


def kernel(x, table, W, b):
    x_flat = x.astype(jnp.int32).reshape(B * L)
    w_rep = jnp.broadcast_to(W.reshape(D, 1), (D, LANES)).reshape(D * LANES)
    out = _tt_call(x_flat, table.reshape(V * D), w_rep,
                   jnp.broadcast_to(b, (LANES,)))
    return out.reshape(B, 1)


# speedup vs baseline: 67.7600x; 1.0158x over previous
import functools

import jax
import jax.numpy as jnp
from jax import lax
from jax.experimental import pallas as pl
from jax.experimental.pallas import tpu as pltpu
from jax.experimental.pallas import tpu_sc as plsc

B = 4096
L = 200
V = 128
D = 32
NC = 2
NS = 16
LANES = 16
NW = NC * NS
BPW = B // NW
G = BPW // LANES


def _sc_body(x_hbm, tab_hbm, w_hbm, b_hbm, out_hbm,
             x_vm, tab_vm, w_vm, b_vm, v_vm, out_vm):
    cid = lax.axis_index("c")
    sid = lax.axis_index("s")
    wid = sid * NC + cid

    pltpu.sync_copy(x_hbm.at[pl.ds(wid * (BPW * L), BPW * L)], x_vm)
    pltpu.sync_copy(tab_hbm, tab_vm)
    pltpu.sync_copy(w_hbm, w_vm)
    pltpu.sync_copy(b_hbm, b_vm)

    lanes = lax.iota(jnp.int32, LANES)

    for g in range(V // LANES):
        rowbase = (g * LANES + lanes) * D
        acc = jnp.zeros((LANES,), jnp.float32)
        for dd in range(D):
            col = plsc.load_gather(tab_vm, [rowbase + dd])
            wsplat = w_vm[pl.ds(dd * LANES, LANES)]
            acc = acc + col * wsplat
        v_vm[pl.ds(g * LANES, LANES)] = acc

    rowoff = lanes * L
    bases = [rowoff + g * (LANES * L) for g in range(G)]

    UNROLL = 8

    def body(jb, accs):
        j0 = jb * UNROLL
        accs = list(accs)
        for u in range(UNROLL):
            for g in range(G):
                idx = bases[g] + (j0 + u)
                xv = plsc.load_gather(x_vm, [idx])
                accs[g] = accs[g] + plsc.load_gather(v_vm, [xv])
        return tuple(accs)

    zero = jnp.zeros((LANES,), jnp.float32)
    accs = lax.fori_loop(0, L // UNROLL, body, tuple(zero for _ in range(G)))

    bvec = b_vm[...]
    for g in range(G):
        z = accs[g] * (1.0 / L) + bvec
        out_vm[pl.ds(g * LANES, LANES)] = 1.0 / (1.0 + jnp.exp(-z))

    pltpu.sync_copy(out_vm, out_hbm.at[pl.ds(wid * BPW, BPW)])


_tt_call = functools.partial(
    pl.kernel,
    out_type=jax.ShapeDtypeStruct((B,), jnp.float32),
    mesh=plsc.VectorSubcoreMesh(core_axis_name="c", subcore_axis_name="s"),
    compiler_params=pltpu.CompilerParams(needs_layout_passes=False),
    scratch_types=[
        pltpu.VMEM((BPW * L,), jnp.int32),
        pltpu.VMEM((V * D,), jnp.float32),
        pltpu.VMEM((D * LANES,), jnp.float32),
        pltpu.VMEM((LANES,), jnp.float32),
        pltpu.VMEM((V,), jnp.float32),
        pltpu.VMEM((BPW,), jnp.float32),
    ],
)(_sc_body)


def kernel(x, table, W, b):
    x_flat = x.astype(jnp.int32).reshape(B * L)
    w_rep = jnp.broadcast_to(W.reshape(D, 1), (D, LANES)).reshape(D * LANES)
    out = _tt_call(x_flat, table.reshape(V * D), w_rep,
                   jnp.broadcast_to(b, (LANES,)))
    return out.reshape(B, 1)
